# baseline (speedup 1.0000x reference)
import jax
import jax.numpy as jnp
from jax import lax
from jax.experimental import pallas as pl
from jax.experimental.pallas import tpu as pltpu
from jax.experimental.pallas import tpu_sc as plsc

N_NODES = 10000
N_PAD = 10240
N_EDGES = 320000
D = 128
K = 128
N_CHUNKS = N_EDGES // K
NC = 2
NS = 16
NW = NC * NS
ROWS_PER_SUB = N_PAD // NS

NPASS = 1
DSUB = D // NPASS

_MESH = plsc.VectorSubcoreMesh(core_axis_name="c", subcore_axis_name="s")


def _zero_vmem_f32(ref, nrows, ncols):
  zero16 = jnp.zeros((16,), jnp.float32)

  def zrow(i, _):
    def zcol(c, _):
      ref[i, pl.ds(c * 16, 16)] = zero16
      return 0
    lax.fori_loop(0, ncols // 16, zcol, 0)
    return 0
  lax.fori_loop(0, nrows, zrow, 0)


def _n_chunks_for(wid):
  return N_CHUNKS // NW + jnp.where(wid < N_CHUNKS % NW, 1, 0)


def _sc_degree():
  scratch = [
      pltpu.VMEM((K,), jnp.int32),
      pltpu.VMEM((N_PAD,), jnp.float32),
  ]

  def body(dst_hbm, deg_out, dst_v, cnt_v):
    cid = lax.axis_index("c")
    sid = lax.axis_index("s")
    wid = sid * NC + cid

    zero16 = jnp.zeros((16,), jnp.float32)

    def zrow(i, _):
      cnt_v[pl.ds(i * 16, 16)] = zero16
      return 0
    lax.fori_loop(0, N_PAD // 16, zrow, 0)

    one16 = jnp.ones((16,), jnp.float32)

    def chunk_body(t, _):
      off = (t * NW + wid) * K
      pltpu.sync_copy(dst_hbm.at[pl.ds(off, K)], dst_v)
      for g in range(K // 16):
        dvec = dst_v[pl.ds(g * 16, 16)]
        plsc.addupdate_scatter(cnt_v, [dvec], one16)
      return 0
    lax.fori_loop(0, _n_chunks_for(wid), chunk_body, 0)

    pltpu.sync_copy(cnt_v, deg_out.at[cid, sid])

  return pl.kernel(
      body,
      out_type=jax.ShapeDtypeStruct((NC, NS, N_PAD), jnp.float32),
      mesh=_MESH,
      scratch_types=scratch,
      compiler_params=pltpu.CompilerParams(needs_layout_passes=False),
  )


NT = 80
E_PAD = NT * NW * K


def _sc_aggregate():
  scratch = [
      pltpu.VMEM((K,), jnp.int32),
      pltpu.VMEM((K,), jnp.int32),
      pltpu.VMEM((K,), jnp.int32),
      pltpu.VMEM((K,), jnp.int32),
      pltpu.VMEM((K, DSUB), jnp.float32),
      pltpu.VMEM((K, DSUB), jnp.float32),
      pltpu.VMEM_SHARED((N_PAD, DSUB), jnp.float32),
      pltpu.SemaphoreType.DMA,
      pltpu.SemaphoreType.DMA,
      pltpu.SemaphoreType.DMA,
      pltpu.SemaphoreType.DMA,
      pltpu.SemaphoreType.DMA,
      pltpu.SemaphoreType.DMA,
  ]

  def body(table, src_hbm, dst_hbm, agg_out, si0, si1, di0, di1, r0, r1,
           agg_sh, i0, i1, g0, g1, s0, s1):
    cid = lax.axis_index("c")
    sid = lax.axis_index("s")
    wid = sid * NC + cid
    base = sid * ROWS_PER_SUB
    ebase = wid * (NT * K)

    _zero_vmem_f32(r0, K, DSUB)
    for j in range(ROWS_PER_SUB // K):
      pltpu.sync_copy(r0, agg_sh.at[pl.ds(base + j * K, K)])

    plsc.subcore_barrier()

    sidx = (si0, si1)
    didx = (di0, di1)
    rows = (r0, r1)
    isem = (i0, i1)
    gsem = (g0, g1)
    ssem = (s0, s1)

    def idx_load(c, b):
      off = ebase + c * K
      pltpu.async_copy(src_hbm.at[pl.ds(off, K)], sidx[b], isem[b])
      pltpu.async_copy(dst_hbm.at[pl.ds(off, K)], didx[b], isem[b])

    def idx_wait(b):
      pltpu.make_async_copy(src_hbm.at[pl.ds(0, K)], sidx[b], isem[b]).wait()
      pltpu.make_async_copy(dst_hbm.at[pl.ds(0, K)], didx[b], isem[b]).wait()

    def gather(b):
      pltpu.async_copy(table.at[sidx[b]], rows[b], gsem[b])

    def gather_wait(b):
      pltpu.make_async_copy(table.at[sidx[b]], rows[b], gsem[b]).wait()

    def scatter(b):
      pltpu.async_copy(rows[b], agg_sh.at[didx[b]], ssem[b], add=True)

    def scatter_wait(b):
      pltpu.make_async_copy(rows[b], agg_sh.at[didx[b]], ssem[b]).wait()

    idx_load(0, 0)
    idx_wait(0)
    gather(0)

    def slot(c, b, first, last):
      gather_wait(b)
      scatter(b)
      if not last:
        nb = 1 - b
        if not first:
          scatter_wait(nb)
        idx_load(c + 1, nb)
        idx_wait(nb)
        gather(nb)

    slot(0, 0, True, False)

    def pipe(t, _):
      c0 = 2 * t + 1
      slot(c0, 1, False, False)
      slot(c0 + 1, 0, False, False)
      return 0
    lax.fori_loop(0, (NT - 2) // 2, pipe, 0)

    slot(NT - 1, 1, False, True)

    for b in range(2):
      scatter_wait(b)

    plsc.subcore_barrier()

    for j in range(ROWS_PER_SUB // K):
      sl = pl.ds(base + j * K, K)
      pltpu.sync_copy(agg_sh.at[sl], r0)
      pltpu.sync_copy(r0, agg_out.at[0, cid, sl])

  return pl.kernel(
      body,
      out_type=jax.ShapeDtypeStruct((NPASS, NC, N_PAD, DSUB), jnp.float32),
      mesh=_MESH,
      scratch_types=scratch,
  )


def _tc_dense(relu: bool, split_out: bool):
  B = 512

  def body(aggp_ref, degp_ref, x_ref, wl_ref, bl_ref, wr_ref, o_ref):
    agg = jnp.concatenate(
        [aggp_ref[p, 0] + aggp_ref[p, 1] for p in range(NPASS)], axis=-1)
    deg = jnp.sum(degp_ref[...], axis=(0, 1))
    deg = jnp.maximum(deg, 1.0)
    mean = agg / deg[:, None]
    x = jnp.concatenate([x_ref[p] for p in range(NPASS)], axis=-1)
    out = (jnp.dot(mean, wl_ref[...], preferred_element_type=jnp.float32)
           + jnp.dot(x, wr_ref[...], preferred_element_type=jnp.float32)
           + bl_ref[...])
    if relu:
      out = jnp.maximum(out, 0.0)
    if split_out:
      for p in range(NPASS):
        o_ref[p] = out[:, p * DSUB:(p + 1) * DSUB]
    else:
      o_ref[...] = out

  if split_out:
    out_spec = pl.BlockSpec((NPASS, B, DSUB), lambda i: (0, i, 0))
    out_shape = jax.ShapeDtypeStruct((NPASS, N_PAD, DSUB), jnp.float32)
  else:
    out_spec = pl.BlockSpec((B, D), lambda i: (i, 0))
    out_shape = jax.ShapeDtypeStruct((N_PAD, D), jnp.float32)

  return pl.pallas_call(
      body,
      grid=(N_PAD // B,),
      in_specs=[
          pl.BlockSpec((NPASS, NC, B, DSUB), lambda i: (0, 0, i, 0)),
          pl.BlockSpec((NC, NS, B), lambda i: (0, 0, i)),
          pl.BlockSpec((NPASS, B, DSUB), lambda i: (0, i, 0)),
          pl.BlockSpec((D, D), lambda i: (0, 0)),
          pl.BlockSpec((1, D), lambda i: (0, 0)),
          pl.BlockSpec((D, D), lambda i: (0, 0)),
      ],
      out_specs=out_spec,
      out_shape=out_shape,
  )


_sc_deg = _sc_degree()
_sc_agg = _sc_aggregate()
_tc_relu_split = _tc_dense(relu=True, split_out=True)
_tc_lin = _tc_dense(relu=False, split_out=False)


@jax.jit
def kernel(x, edge_index, Wl1, bl1, Wr1, Wl2, bl2, Wr2):
  src = edge_index[0].astype(jnp.int32)
  dst = edge_index[1].astype(jnp.int32)
  xp = jnp.zeros((N_PAD, D), jnp.float32).at[:N_NODES].set(x)
  xg = xp.reshape(N_PAD, NPASS, DSUB).transpose(1, 0, 2)

  n_extra = E_PAD - N_EDGES
  src_p = jnp.concatenate([src, jnp.zeros((n_extra,), jnp.int32)])
  dst_p = jnp.concatenate(
      [dst,
       N_NODES + (jnp.arange(n_extra, dtype=jnp.int32) % (N_PAD - N_NODES))])

  degp = _sc_deg(dst)
  agg1 = _sc_agg(*[xg[p] for p in range(NPASS)], src_p, dst_p)
  hg = _tc_relu_split(agg1, degp, xg, Wl1, bl1.reshape(1, D), Wr1)
  agg2 = _sc_agg(*[hg[p] for p in range(NPASS)], src_p, dst_p)
  out = _tc_lin(agg2, degp, hg, Wl2, bl2.reshape(1, D), Wr2)
  return out[:N_NODES]


# speedup vs baseline: 2.6749x; 2.6749x over previous
import jax
import jax.numpy as jnp
from jax import lax
from jax.experimental import pallas as pl
from jax.experimental.pallas import tpu as pltpu
from jax.experimental.pallas import tpu_sc as plsc

N_NODES = 10000
N_PAD = 10240
N_EDGES = 320000
D = 128
K = 128
N_CHUNKS = N_EDGES // K
NC = 2
NS = 16
NW = NC * NS
ROWS_PER_SUB = N_PAD // NS

NPASS = 1
DSUB = D // NPASS

_MESH = plsc.VectorSubcoreMesh(core_axis_name="c", subcore_axis_name="s")


def _zero_vmem_f32(ref, nrows, ncols):
  zero16 = jnp.zeros((16,), jnp.float32)

  def zrow(i, _):
    def zcol(c, _):
      ref[i, pl.ds(c * 16, 16)] = zero16
      return 0
    lax.fori_loop(0, ncols // 16, zcol, 0)
    return 0
  lax.fori_loop(0, nrows, zrow, 0)


def _n_chunks_for(wid):
  return N_CHUNKS // NW + jnp.where(wid < N_CHUNKS % NW, 1, 0)


def _sc_degree():
  scratch = [
      pltpu.VMEM((K,), jnp.int32),
      pltpu.VMEM((N_PAD,), jnp.float32),
  ]

  def body(dst_hbm, deg_out, dst_v, cnt_v):
    cid = lax.axis_index("c")
    sid = lax.axis_index("s")
    wid = sid * NC + cid

    zero16 = jnp.zeros((16,), jnp.float32)

    def zrow(i, _):
      cnt_v[pl.ds(i * 16, 16)] = zero16
      return 0
    lax.fori_loop(0, N_PAD // 16, zrow, 0)

    one16 = jnp.ones((16,), jnp.float32)

    def chunk_body(t, _):
      off = (t * NW + wid) * K
      pltpu.sync_copy(dst_hbm.at[pl.ds(off, K)], dst_v)
      for g in range(K // 16):
        dvec = dst_v[pl.ds(g * 16, 16)]
        plsc.addupdate_scatter(cnt_v, [dvec], one16)
      return 0
    lax.fori_loop(0, _n_chunks_for(wid), chunk_body, 0)

    pltpu.sync_copy(cnt_v, deg_out.at[cid, sid])

  return pl.kernel(
      body,
      out_type=jax.ShapeDtypeStruct((NC, NS, N_PAD), jnp.float32),
      mesh=_MESH,
      scratch_types=scratch,
      compiler_params=pltpu.CompilerParams(needs_layout_passes=False),
  )


def _sc_aggregate():
  scratch = [
      pltpu.VMEM((K,), jnp.int32),
      pltpu.VMEM((K,), jnp.int32),
      pltpu.VMEM((K,), jnp.int32),
      pltpu.VMEM((K, DSUB), jnp.float32),
      pltpu.VMEM((K, DSUB), jnp.float32),
      pltpu.VMEM_SHARED((N_PAD, DSUB), jnp.float32),
      pltpu.SemaphoreType.DMA,
      pltpu.SemaphoreType.DMA,
      pltpu.SemaphoreType.DMA,
      pltpu.SemaphoreType.DMA,
  ]

  def body(*args):
    tables = args[:NPASS]
    (src_hbm, dst_hbm, agg_out, src_v, di0, di1, r0, r1, agg_sh,
     isem, gsem, s0, s1) = args[NPASS:]
    cid = lax.axis_index("c")
    sid = lax.axis_index("s")
    wid = sid * NC + cid
    base = sid * ROWS_PER_SUB
    n_t = _n_chunks_for(wid)

    didx = (di0, di1)
    rows = (r0, r1)
    ssem = (s0, s1)

    for p in range(NPASS):
      _zero_vmem_f32(r0, K, DSUB)
      for j in range(ROWS_PER_SUB // K):
        pltpu.sync_copy(r0, agg_sh.at[pl.ds(base + j * K, K)])

      plsc.subcore_barrier()

      def do_chunk(t, b):
        off = (t * NW + wid) * K
        pltpu.async_copy(src_hbm.at[pl.ds(off, K)], src_v, isem)

        @pl.when(t >= 2)
        def _():
          pltpu.make_async_copy(rows[b], agg_sh.at[didx[b]],
                                ssem[b]).wait()
        pltpu.async_copy(dst_hbm.at[pl.ds(off, K)], didx[b], isem)
        pltpu.make_async_copy(src_hbm.at[pl.ds(off, K)], src_v, isem).wait()
        pltpu.async_copy(tables[p].at[src_v], rows[b], gsem).wait()
        pltpu.make_async_copy(dst_hbm.at[pl.ds(off, K)], didx[b],
                              isem).wait()
        pltpu.async_copy(rows[b], agg_sh.at[didx[b]], ssem[b], add=True)

      def chunk_body(t, _):
        @pl.when(t % 2 == 0)
        def _():
          do_chunk(t, 0)

        @pl.when(t % 2 == 1)
        def _():
          do_chunk(t, 1)
        return 0
      lax.fori_loop(0, n_t, chunk_body, 0)

      for b in range(2):
        pltpu.make_async_copy(rows[b], agg_sh.at[didx[b]], ssem[b]).wait()

      plsc.subcore_barrier()

      for j in range(ROWS_PER_SUB // K):
        sl = pl.ds(base + j * K, K)
        pltpu.sync_copy(agg_sh.at[sl], r0)
        pltpu.sync_copy(r0, agg_out.at[p, cid, sl])

      if p + 1 < NPASS:
        plsc.subcore_barrier()

  return pl.kernel(
      body,
      out_type=jax.ShapeDtypeStruct((NPASS, NC, N_PAD, DSUB), jnp.float32),
      mesh=_MESH,
      scratch_types=scratch,
  )


def _tc_dense(relu: bool, split_out: bool):
  B = 512

  def body(aggp_ref, degp_ref, x_ref, wl_ref, bl_ref, wr_ref, o_ref):
    agg = jnp.concatenate(
        [aggp_ref[p, 0] + aggp_ref[p, 1] for p in range(NPASS)], axis=-1)
    deg = jnp.sum(degp_ref[...], axis=(0, 1))
    deg = jnp.maximum(deg, 1.0)
    mean = agg / deg[:, None]
    x = jnp.concatenate([x_ref[p] for p in range(NPASS)], axis=-1)
    out = (jnp.dot(mean, wl_ref[...], preferred_element_type=jnp.float32)
           + jnp.dot(x, wr_ref[...], preferred_element_type=jnp.float32)
           + bl_ref[...])
    if relu:
      out = jnp.maximum(out, 0.0)
    if split_out:
      for p in range(NPASS):
        o_ref[p] = out[:, p * DSUB:(p + 1) * DSUB]
    else:
      o_ref[...] = out

  if split_out:
    out_spec = pl.BlockSpec((NPASS, B, DSUB), lambda i: (0, i, 0))
    out_shape = jax.ShapeDtypeStruct((NPASS, N_PAD, DSUB), jnp.float32)
  else:
    out_spec = pl.BlockSpec((B, D), lambda i: (i, 0))
    out_shape = jax.ShapeDtypeStruct((N_PAD, D), jnp.float32)

  return pl.pallas_call(
      body,
      grid=(N_PAD // B,),
      in_specs=[
          pl.BlockSpec((NPASS, NC, B, DSUB), lambda i: (0, 0, i, 0)),
          pl.BlockSpec((NC, NS, B), lambda i: (0, 0, i)),
          pl.BlockSpec((NPASS, B, DSUB), lambda i: (0, i, 0)),
          pl.BlockSpec((D, D), lambda i: (0, 0)),
          pl.BlockSpec((1, D), lambda i: (0, 0)),
          pl.BlockSpec((D, D), lambda i: (0, 0)),
      ],
      out_specs=out_spec,
      out_shape=out_shape,
  )


_sc_deg = _sc_degree()
_sc_agg = _sc_aggregate()
_tc_relu_split = _tc_dense(relu=True, split_out=True)
_tc_lin = _tc_dense(relu=False, split_out=False)


@jax.jit
def kernel(x, edge_index, Wl1, bl1, Wr1, Wl2, bl2, Wr2):
  src = edge_index[0].astype(jnp.int32)
  dst = edge_index[1].astype(jnp.int32)
  xp = jnp.zeros((N_PAD, D), jnp.float32).at[:N_NODES].set(x)
  xg = xp.reshape(N_PAD, NPASS, DSUB).transpose(1, 0, 2)

  degp = _sc_deg(dst)
  agg1 = _sc_agg(*[xg[p] for p in range(NPASS)], src, dst)
  hg = _tc_relu_split(agg1, degp, xg, Wl1, bl1.reshape(1, D), Wr1)
  agg2 = _sc_agg(*[hg[p] for p in range(NPASS)], src, dst)
  out = _tc_lin(agg2, degp, hg, Wl2, bl2.reshape(1, D), Wr2)
  return out[:N_NODES]
